# baseline (speedup 1.0000x reference)
import dataclasses
import functools

import jax
import jax.numpy as jnp
from jax import lax
from jax.experimental import pallas as pl
from jax.experimental.pallas import tpu as pltpu
from jax.experimental.pallas import tpu_sc as plsc

_N = 8 * 512 * 512
_ROWS = 4096
_COLS = 512

_SC_ROWS = 256
_SC_BLK_ROWS = 8
_SC_GRID = _SC_ROWS // _SC_BLK_ROWS
_SC_OFF = (_ROWS - _SC_ROWS) // _SC_BLK_ROWS

_TC_ROWS = _ROWS - _SC_ROWS
_TC_STEPS = 2
_TC_BLK_ROWS = _TC_ROWS // _TC_STEPS

_NW = 32
_L = 16

_LN2 = 0.6931471805599453
_SQRT2 = 1.4142135623730951


def _softlog(x):
    xi = plsc.bitcast(x, jnp.int32)
    e = (xi >> 23) - 127
    m = plsc.bitcast((xi & 0x7FFFFF) | 0x3F800000, jnp.float32)
    big = m > _SQRT2
    m = jnp.where(big, m * 0.5, m)
    e = jnp.where(big, e + 1, e)
    s = (m - 1.0) / (m + 1.0)
    s2 = s * s
    poly = 2.0 * s * (1.0 + s2 * (1 / 3 + s2 * (1 / 5 + s2 * (1 / 7 + s2 * (1 / 9)))))
    return e.astype(jnp.float32) * _LN2 + poly


def _focal_vec(xv, tv):
    p = 1.0 / (1.0 + jnp.exp(-xv))
    prob = jnp.where(tv == 1, p, 1.0 - p)
    prob = jnp.clip(prob, 1e-8, 1.0 - 1e-8)
    om = 1.0 - prob
    return om * om * (-_softlog(prob))


def _sc_body(x_hbm, t_hbm, o_hbm, acc_v, sem):
    acc_v[...] = jnp.zeros((_L,), jnp.float32)

    def body(x_vmem, t_vmem):
        @pl.loop(0, _SC_BLK_ROWS)
        def _(r):
            @pl.loop(0, _COLS, step=_L)
            def _(c):
                xv = x_vmem[r, pl.ds(c, _L)]
                tv = t_vmem[r, pl.ds(c, _L)]
                acc_v[...] = acc_v[...] + _focal_vec(xv, tv)

    pltpu.emit_pipeline(
        body,
        grid=(_SC_GRID,),
        in_specs=[
            pl.BlockSpec((_SC_BLK_ROWS, _COLS), lambda i: (i + _SC_OFF, 0)),
            pl.BlockSpec((_SC_BLK_ROWS, _COLS), lambda i: (i + _SC_OFF, 0)),
        ],
        core_axis_name=("c", "s"),
        dimension_semantics=(pltpu.PARALLEL,),
    )(x_hbm, t_hbm)

    wid = lax.axis_index("s") * 2 + lax.axis_index("c")
    pltpu.sync_copy(acc_v, o_hbm.at[pl.ds(wid * _L, _L)])


_sc_compiler_params = pltpu.CompilerParams()
if "needs_layout_passes" in pltpu.CompilerParams.__dataclass_fields__:
    _sc_compiler_params = dataclasses.replace(
        _sc_compiler_params, needs_layout_passes=False
    )

_sc_call = functools.partial(
    pl.kernel,
    out_type=jax.ShapeDtypeStruct((_NW * _L,), jnp.float32),
    mesh=plsc.VectorSubcoreMesh(core_axis_name="c", subcore_axis_name="s"),
    scratch_types=[pltpu.VMEM((_L,), jnp.float32), pltpu.SemaphoreType.DMA],
    compiler_params=_sc_compiler_params,
)(_sc_body)


def _tc_body(x_ref, t_ref, o_ref):
    i = pl.program_id(0)
    x = x_ref[...]
    t = t_ref[...]
    p = jax.nn.sigmoid(x)
    prob = jnp.where(t == 1, p, 1.0 - p)
    prob = jnp.clip(prob, 1e-8, 1.0 - 1e-8)
    om = 1.0 - prob
    s = jnp.sum(om * om * (-jnp.log(prob)))

    @pl.when(i == 0)
    def _():
        o_ref[0, 0] = s

    @pl.when(i > 0)
    def _():
        o_ref[0, 0] = o_ref[0, 0] + s


def kernel(logit, target):
    x = logit.reshape(_ROWS, _COLS)
    t = target.reshape(_ROWS, _COLS).astype(jnp.int32)
    sc_part = _sc_call(x, t)
    tc_part = pl.pallas_call(
        _tc_body,
        grid=(_TC_STEPS,),
        in_specs=[
            pl.BlockSpec((_TC_BLK_ROWS, _COLS), lambda i: (i, 0)),
            pl.BlockSpec((_TC_BLK_ROWS, _COLS), lambda i: (i, 0)),
        ],
        out_specs=pl.BlockSpec(memory_space=pltpu.MemorySpace.SMEM),
        out_shape=jax.ShapeDtypeStruct((1, 1), jnp.float32),
        compiler_params=pltpu.CompilerParams(
            dimension_semantics=("arbitrary",),
        ),
    )(x, t)
    return (tc_part[0, 0] + jnp.sum(sc_part)) * (1.0 / _N)


# speedup vs baseline: 3.8291x; 3.8291x over previous
import jax
import jax.numpy as jnp
from jax.experimental import pallas as pl
from jax.experimental.pallas import tpu as pltpu

_N = 8 * 512 * 512
_ROWS = 4096
_COLS = 512
_K = 4
_STEPS = 4
_SROWS = _ROWS // _K
_BROWS = _SROWS // _STEPS


def _focal_sum(x, t):
    p = jax.nn.sigmoid(x)
    prob = jnp.where(t == 1, p, 1.0 - p)
    prob = jnp.clip(prob, 1e-8, 1.0 - 1e-8)
    om = 1.0 - prob
    return jnp.sum(om * om * (-jnp.log(prob)))


def _tc_body(*refs):
    o_ref = refs[-1]
    i = pl.program_id(0)
    s = _focal_sum(refs[0][...], refs[_K][...])
    for j in range(1, _K):
        s = s + _focal_sum(refs[j][...], refs[_K + j][...])

    @pl.when(i == 0)
    def _():
        o_ref[0, 0] = s

    @pl.when(i > 0)
    def _():
        o_ref[0, 0] = o_ref[0, 0] + s

    @pl.when(i == _STEPS - 1)
    def _():
        o_ref[0, 0] = o_ref[0, 0] * (1.0 / _N)


def _spec(j):
    return pl.BlockSpec((_BROWS, _COLS), lambda i, j=j: (j * _STEPS + i, 0))


def kernel(logit, target):
    x = logit.reshape(_ROWS, _COLS)
    t = target.reshape(_ROWS, _COLS).astype(jnp.int32)
    out = pl.pallas_call(
        _tc_body,
        grid=(_STEPS,),
        in_specs=[_spec(j) for j in range(_K)] * 2,
        out_specs=pl.BlockSpec(memory_space=pltpu.MemorySpace.SMEM),
        out_shape=jax.ShapeDtypeStruct((1, 1), jnp.float32),
        compiler_params=pltpu.CompilerParams(
            dimension_semantics=("arbitrary",),
        ),
    )(*([x] * _K + [t] * _K))
    return out.reshape(())
